# Initial kernel scaffold; baseline (speedup 1.0000x reference)
import jax
import jax.numpy as jnp
from jax.experimental import pallas as pl


def kernel(w, beta, x, y, particle_id):
    raise NotImplementedError("write your pallas kernel here")


# speedup vs baseline: 2.4109x; 2.4109x over previous
import functools

import jax
import jax.numpy as jnp
from jax.experimental import pallas as pl
from jax.experimental.pallas import tpu as pltpu

_N = 8192
_D = 16
_P = 256
_N_BLK = 1024
_NB = _N // _N_BLK
_Q_MIN = 0.01
_REP = 10.0


def _phases_kernel(beta_ref, pid_ref, x_ref, out_ref,
                   bestq, bestn, xat, qa, racc, tacc):
    ph = pl.program_id(0)
    b = pl.program_id(1)

    pid_col = pid_ref[...]
    beta_col = beta_ref[...]
    x_blk = x_ref[...]

    at = 0.5 * jnp.log((1.0 + beta_col) / (1.0 - beta_col))
    q_col = at * at + _Q_MIN

    lane = jax.lax.broadcasted_iota(jnp.int32, (_N_BLK, _P), 1)
    mask = pid_col == (lane + 1)
    n_iota = jax.lax.broadcasted_iota(jnp.int32, (_N_BLK, _P), 0) + b * _N_BLK

    @pl.when(jnp.logical_and(ph == 0, b == 0))
    def _init():
        bestq[...] = jnp.full((1, _P), -1.0, jnp.float32)
        bestn[...] = jnp.zeros((1, _P), jnp.int32)
        xat[...] = jnp.zeros((_D, _P), jnp.float32)
        qa[...] = jnp.zeros((1, _P), jnp.float32)
        racc[...] = jnp.zeros((1, _P), jnp.float32)
        tacc[...] = jnp.zeros((1, _P), jnp.float32)

    @pl.when(ph == 0)
    def _phase0():
        mq = jnp.where(mask, q_col, -1.0)
        bmax = jnp.max(mq, axis=0, keepdims=True)
        nidx = jnp.where(mq == bmax, n_iota, _N)
        bmin = jnp.min(nidx, axis=0, keepdims=True)
        upd = bmax > bestq[...]
        bestq[...] = jnp.where(upd, bmax, bestq[...])
        bestn[...] = jnp.where(upd, bmin, bestn[...])

    @pl.when(ph == 1)
    def _phase1():
        onehot = (n_iota == bestn[...]).astype(jnp.float32)
        xat[...] += jax.lax.dot_general(
            x_blk, onehot, (((0,), (0,)), ((), ())),
            preferred_element_type=jnp.float32)
        qa[...] += jax.lax.dot_general(
            q_col, onehot, (((0,), (0,)), ((), ())),
            preferred_element_type=jnp.float32)

    @pl.when(ph == 2)
    def _phase2():
        xa = xat[...]
        dot = jax.lax.dot_general(
            x_blk, xa, (((1,), (0,)), ((), ())),
            preferred_element_type=jnp.float32)
        xn2 = jnp.sum(x_blk * x_blk, axis=1, keepdims=True)
        xa2 = jnp.sum(xa * xa, axis=0, keepdims=True)
        d2 = jnp.maximum(xn2 + xa2 - 2.0 * dot, 0.0)
        hinge = jnp.maximum(1.0 - jnp.sqrt(d2), 0.0)
        qh = q_col * hinge
        racc[...] += jnp.sum(qh, axis=0, keepdims=True)
        seg = jnp.where(mask, q_col * d2 - _REP * qh, 0.0)
        tacc[...] += jnp.sum(seg, axis=0, keepdims=True)

        @pl.when(b == _NB - 1)
        def _final():
            present = (bestq[...] >= 0.0).astype(jnp.float32)
            s = qa[...] * (tacc[...] + _REP * racc[...]) * present
            out_ref[...] = jnp.sum(s, axis=(0, 1), keepdims=True) / _N


@functools.partial(jax.jit)
def _potential_loss(beta, x, particle_id):
    beta2 = beta.reshape(_N, 1)
    pid2 = particle_id.reshape(_N, 1)
    out = pl.pallas_call(
        _phases_kernel,
        grid=(3, _NB),
        in_specs=[
            pl.BlockSpec((_N_BLK, 1), lambda ph, b: (b, 0)),
            pl.BlockSpec((_N_BLK, 1), lambda ph, b: (b, 0)),
            pl.BlockSpec((_N_BLK, _D), lambda ph, b: (b, 0)),
        ],
        out_specs=pl.BlockSpec((1, 1), lambda ph, b: (0, 0)),
        out_shape=jax.ShapeDtypeStruct((1, 1), jnp.float32),
        scratch_shapes=[
            pltpu.VMEM((1, _P), jnp.float32),
            pltpu.VMEM((1, _P), jnp.int32),
            pltpu.VMEM((_D, _P), jnp.float32),
            pltpu.VMEM((1, _P), jnp.float32),
            pltpu.VMEM((1, _P), jnp.float32),
            pltpu.VMEM((1, _P), jnp.float32),
        ],
        compiler_params=pltpu.CompilerParams(
            dimension_semantics=("arbitrary", "arbitrary"),
        ),
    )(beta2, pid2, x)
    return out[0, 0]


def kernel(w, beta, x, y, particle_id):
    return _potential_loss(beta, x, particle_id)


# speedup vs baseline: 3.0679x; 1.2725x over previous
import functools

import jax
import jax.numpy as jnp
from jax.experimental import pallas as pl
from jax.experimental.pallas import tpu as pltpu

_N = 8192
_D = 16
_P = 256
_N_BLK = 1024
_NB = _N // _N_BLK
_Q_MIN = 0.01
_REP = 10.0


def _select_kernel(beta_ref, pid_ref, x_ref, q_out, xat_out, bestq_out):
    b = pl.program_id(0)

    @pl.when(b == 0)
    def _init():
        xat_out[...] = jnp.zeros((_D, _P), jnp.float32)
        bestq_out[...] = jnp.full((1, _P), -1.0, jnp.float32)

    beta_col = beta_ref[...]
    at = 0.5 * jnp.log((1.0 + beta_col) / (1.0 - beta_col))
    q_col = at * at + _Q_MIN
    q_out[...] = q_col

    lane = jax.lax.broadcasted_iota(jnp.int32, (_N_BLK, _P), 1)
    mask = pid_ref[...] == (lane + 1)
    n_loc = jax.lax.broadcasted_iota(jnp.int32, (_N_BLK, _P), 0)

    mq = jnp.where(mask, q_col, -1.0)
    bmax = jnp.max(mq, axis=0, keepdims=True)
    nidx = jnp.where(mq == bmax, n_loc, _N)
    bmin = jnp.min(nidx, axis=0, keepdims=True)
    upd = bmax > bestq_out[...]

    sel = jnp.logical_and(n_loc == bmin, upd).astype(jnp.float32)
    xcand = jax.lax.dot_general(
        x_ref[...], sel, (((0,), (0,)), ((), ())),
        preferred_element_type=jnp.float32)
    xat_out[...] = jnp.where(upd, xcand, xat_out[...])
    bestq_out[...] = jnp.where(upd, bmax, bestq_out[...])


def _dense_kernel(q_ref, pid_ref, x_ref, xat_ref, bestq_ref, out_ref,
                  racc, tacc):
    b = pl.program_id(0)

    @pl.when(b == 0)
    def _init():
        racc[...] = jnp.zeros((1, _P), jnp.float32)
        tacc[...] = jnp.zeros((1, _P), jnp.float32)

    q_col = q_ref[...]
    x_blk = x_ref[...]
    xa = xat_ref[...]

    dot = jax.lax.dot_general(
        x_blk, xa, (((1,), (0,)), ((), ())),
        preferred_element_type=jnp.float32)
    xn2 = jnp.sum(x_blk * x_blk, axis=1, keepdims=True)
    xa2 = jnp.sum(xa * xa, axis=0, keepdims=True)
    d2 = jnp.maximum(xn2 + xa2 - 2.0 * dot, 0.0)
    hinge = jnp.maximum(1.0 - jnp.sqrt(d2), 0.0)

    lane = jax.lax.broadcasted_iota(jnp.int32, (_N_BLK, _P), 1)
    mask = pid_ref[...] == (lane + 1)
    seg = jnp.where(mask, d2 - _REP * hinge, 0.0)

    racc[...] += jax.lax.dot_general(
        q_col, hinge, (((0,), (0,)), ((), ())),
        preferred_element_type=jnp.float32)
    tacc[...] += jax.lax.dot_general(
        q_col, seg, (((0,), (0,)), ((), ())),
        preferred_element_type=jnp.float32)

    @pl.when(b == _NB - 1)
    def _final():
        bq = bestq_ref[...]
        present = (bq >= 0.0).astype(jnp.float32)
        s = bq * (tacc[...] + _REP * racc[...]) * present
        out_ref[...] = jnp.sum(s, axis=(0, 1), keepdims=True) / _N


@functools.partial(jax.jit)
def _potential_loss(beta, x, particle_id):
    beta2 = beta.reshape(_N, 1)
    pid2 = particle_id.reshape(_N, 1)

    q2, xat, bestq = pl.pallas_call(
        _select_kernel,
        grid=(_NB,),
        in_specs=[
            pl.BlockSpec((_N_BLK, 1), lambda b: (b, 0)),
            pl.BlockSpec((_N_BLK, 1), lambda b: (b, 0)),
            pl.BlockSpec((_N_BLK, _D), lambda b: (b, 0)),
        ],
        out_specs=[
            pl.BlockSpec((_N_BLK, 1), lambda b: (b, 0)),
            pl.BlockSpec((_D, _P), lambda b: (0, 0)),
            pl.BlockSpec((1, _P), lambda b: (0, 0)),
        ],
        out_shape=[
            jax.ShapeDtypeStruct((_N, 1), jnp.float32),
            jax.ShapeDtypeStruct((_D, _P), jnp.float32),
            jax.ShapeDtypeStruct((1, _P), jnp.float32),
        ],
        compiler_params=pltpu.CompilerParams(
            dimension_semantics=("arbitrary",),
        ),
    )(beta2, pid2, x)

    out = pl.pallas_call(
        _dense_kernel,
        grid=(_NB,),
        in_specs=[
            pl.BlockSpec((_N_BLK, 1), lambda b: (b, 0)),
            pl.BlockSpec((_N_BLK, 1), lambda b: (b, 0)),
            pl.BlockSpec((_N_BLK, _D), lambda b: (b, 0)),
            pl.BlockSpec((_D, _P), lambda b: (0, 0)),
            pl.BlockSpec((1, _P), lambda b: (0, 0)),
        ],
        out_specs=pl.BlockSpec((1, 1), lambda b: (0, 0)),
        out_shape=jax.ShapeDtypeStruct((1, 1), jnp.float32),
        scratch_shapes=[
            pltpu.VMEM((1, _P), jnp.float32),
            pltpu.VMEM((1, _P), jnp.float32),
        ],
        compiler_params=pltpu.CompilerParams(
            dimension_semantics=("arbitrary",),
        ),
    )(q2, pid2, x, xat, bestq)
    return out[0, 0]


def kernel(w, beta, x, y, particle_id):
    return _potential_loss(beta, x, particle_id)


# speedup vs baseline: 3.4179x; 1.1141x over previous
import functools

import jax
import jax.numpy as jnp
from jax.experimental import pallas as pl
from jax.experimental.pallas import tpu as pltpu

_N = 8192
_D = 16
_P = 256
_N_BLK = 2048
_NB = _N // _N_BLK
_Q_MIN = 0.01
_REP = 10.0


def _select_kernel(beta_ref, pid_ref, x_ref, q_out, xat_out, bestq_out):
    b = pl.program_id(0)

    @pl.when(b == 0)
    def _init():
        xat_out[...] = jnp.zeros((_D, _P), jnp.float32)
        bestq_out[...] = jnp.full((1, _P), -1.0, jnp.float32)

    beta_col = beta_ref[...]
    at = 0.5 * jnp.log((1.0 + beta_col) / (1.0 - beta_col))
    q_col = at * at + _Q_MIN
    q_out[...] = q_col

    lane = jax.lax.broadcasted_iota(jnp.int32, (_N_BLK, _P), 1)
    mask = pid_ref[...] == (lane + 1)
    n_loc = jax.lax.broadcasted_iota(jnp.int32, (_N_BLK, _P), 0)

    mq = jnp.where(mask, q_col, -1.0)
    bmax = jnp.max(mq, axis=0, keepdims=True)
    nidx = jnp.where(mq == bmax, n_loc, _N)
    bmin = jnp.min(nidx, axis=0, keepdims=True)
    upd = bmax > bestq_out[...]

    sel = jnp.logical_and(n_loc == bmin, upd).astype(jnp.float32)
    xcand = jax.lax.dot_general(
        x_ref[...], sel, (((0,), (0,)), ((), ())),
        preferred_element_type=jnp.float32)
    xat_out[...] = jnp.where(upd, xcand, xat_out[...])
    bestq_out[...] = jnp.where(upd, bmax, bestq_out[...])


def _dense_kernel(q_ref, pid_ref, x_ref, xat_ref, bestq_ref, out_ref,
                  racc, tacc):
    b = pl.program_id(0)

    @pl.when(b == 0)
    def _init():
        racc[...] = jnp.zeros((1, _P), jnp.float32)
        tacc[...] = jnp.zeros((1, _P), jnp.float32)

    q_col = q_ref[...]
    x_blk = x_ref[...]
    xa = xat_ref[...]

    dot = jax.lax.dot_general(
        x_blk, xa, (((1,), (0,)), ((), ())),
        preferred_element_type=jnp.float32)
    xn2 = jnp.sum(x_blk * x_blk, axis=1, keepdims=True)
    xa2 = jnp.sum(xa * xa, axis=0, keepdims=True)
    d2 = jnp.maximum(xn2 + xa2 - 2.0 * dot, 0.0)
    hinge = jnp.maximum(1.0 - jnp.sqrt(d2), 0.0)

    lane = jax.lax.broadcasted_iota(jnp.int32, (_N_BLK, _P), 1)
    mask = pid_ref[...] == (lane + 1)
    seg = jnp.where(mask, d2 - _REP * hinge, 0.0)

    racc[...] += jax.lax.dot_general(
        q_col, hinge, (((0,), (0,)), ((), ())),
        preferred_element_type=jnp.float32)
    tacc[...] += jax.lax.dot_general(
        q_col, seg, (((0,), (0,)), ((), ())),
        preferred_element_type=jnp.float32)

    @pl.when(b == _NB - 1)
    def _final():
        bq = bestq_ref[...]
        present = (bq >= 0.0).astype(jnp.float32)
        s = bq * (tacc[...] + _REP * racc[...]) * present
        out_ref[...] = jnp.sum(s, axis=(0, 1), keepdims=True) / _N


@functools.partial(jax.jit)
def _potential_loss(beta, x, particle_id):
    beta2 = beta.reshape(_N, 1)
    pid2 = particle_id.reshape(_N, 1)

    q2, xat, bestq = pl.pallas_call(
        _select_kernel,
        grid=(_NB,),
        in_specs=[
            pl.BlockSpec((_N_BLK, 1), lambda b: (b, 0)),
            pl.BlockSpec((_N_BLK, 1), lambda b: (b, 0)),
            pl.BlockSpec((_N_BLK, _D), lambda b: (b, 0)),
        ],
        out_specs=[
            pl.BlockSpec((_N_BLK, 1), lambda b: (b, 0)),
            pl.BlockSpec((_D, _P), lambda b: (0, 0)),
            pl.BlockSpec((1, _P), lambda b: (0, 0)),
        ],
        out_shape=[
            jax.ShapeDtypeStruct((_N, 1), jnp.float32),
            jax.ShapeDtypeStruct((_D, _P), jnp.float32),
            jax.ShapeDtypeStruct((1, _P), jnp.float32),
        ],
        compiler_params=pltpu.CompilerParams(
            dimension_semantics=("arbitrary",),
        ),
    )(beta2, pid2, x)

    out = pl.pallas_call(
        _dense_kernel,
        grid=(_NB,),
        in_specs=[
            pl.BlockSpec((_N_BLK, 1), lambda b: (b, 0)),
            pl.BlockSpec((_N_BLK, 1), lambda b: (b, 0)),
            pl.BlockSpec((_N_BLK, _D), lambda b: (b, 0)),
            pl.BlockSpec((_D, _P), lambda b: (0, 0)),
            pl.BlockSpec((1, _P), lambda b: (0, 0)),
        ],
        out_specs=pl.BlockSpec((1, 1), lambda b: (0, 0)),
        out_shape=jax.ShapeDtypeStruct((1, 1), jnp.float32),
        scratch_shapes=[
            pltpu.VMEM((1, _P), jnp.float32),
            pltpu.VMEM((1, _P), jnp.float32),
        ],
        compiler_params=pltpu.CompilerParams(
            dimension_semantics=("arbitrary",),
        ),
    )(q2, pid2, x, xat, bestq)
    return out[0, 0]


def kernel(w, beta, x, y, particle_id):
    return _potential_loss(beta, x, particle_id)
